# baseline (speedup 1.0000x reference)
import functools

import jax
import jax.numpy as jnp
from jax import lax
from jax.experimental import pallas as pl
from jax.experimental.pallas import tpu as pltpu
from jax.experimental.pallas import tpu_sc as plsc

NC = 2
NS = 16
NW = NC * NS
CHUNK = 128


def _sc_gather(idx3, table0, table1, B, D, n_chunks, half):
    mesh = plsc.VectorSubcoreMesh(core_axis_name="c", subcore_axis_name="s")
    b_per_w = n_chunks * CHUNK

    @functools.partial(
        pl.kernel,
        mesh=mesh,
        out_type=jax.ShapeDtypeStruct((B, D), jnp.float32),
        compiler_params=pltpu.CompilerParams(use_tc_tiling_on_sc=False),
        scratch_types=[
            pltpu.VMEM((n_chunks, CHUNK), jnp.int32),
            pltpu.VMEM((n_chunks, CHUNK), jnp.int32),
            pltpu.VMEM((n_chunks, CHUNK), jnp.int32),
            pltpu.VMEM((n_chunks, CHUNK, D), jnp.float32),
            pltpu.VMEM((n_chunks, CHUNK, D), jnp.float32),
            pltpu.SemaphoreType.DMA,
            pltpu.SemaphoreType.DMA,
        ],
    )
    def k(idx_hbm, t0_hbm, t1_hbm, out_hbm,
          idx_v, idx0_v, idx1_v, rows0_v, rows1_v, gsem, ssem):
        wid = lax.axis_index("s") * NC + lax.axis_index("c")
        base = wid * b_per_w
        pltpu.sync_copy(idx_hbm.at[wid], idx_v)

        for c in range(n_chunks):
            for g in range(CHUNK // 16):
                v = idx_v[c, pl.ds(g * 16, 16)]
                in1 = v >= half
                v0 = jnp.where(in1, 0, v)
                v1 = jnp.where(in1, v - half, 0)
                idx0_v[c, pl.ds(g * 16, 16)] = v0
                idx1_v[c, pl.ds(g * 16, 16)] = v1

        gathers = []
        for c in range(n_chunks):
            gathers.append(
                pltpu.async_copy(t0_hbm.at[idx0_v.at[c]], rows0_v.at[c], gsem)
            )
            gathers.append(
                pltpu.async_copy(t1_hbm.at[idx1_v.at[c]], rows1_v.at[c], gsem)
            )
        for g in gathers:
            g.wait()

        for c in range(n_chunks):
            def merge_body(g, _, c=c):
                sel = idx_v[c, pl.ds(g * 16, 16)]
                for j in range(16):
                    in1 = (
                        lax.squeeze(lax.slice_in_dim(sel, j, j + 1), (0,))
                        >= half
                    )
                    i = g * 16 + j
                    for q in range(D // 16):
                        a = rows0_v[c, i, pl.ds(q * 16, 16)]
                        b = rows1_v[c, i, pl.ds(q * 16, 16)]
                        rows0_v[c, i, pl.ds(q * 16, 16)] = jnp.where(
                            in1, b, a
                        )
                return ()

            lax.fori_loop(0, CHUNK // 16, merge_body, ())

        for c in range(n_chunks):
            pltpu.async_copy(
                rows0_v.at[c], out_hbm.at[pl.ds(base + c * CHUNK, CHUNK)], ssem
            ).wait()

    return k(idx3, table0, table1)


def kernel(drug_ids, table):
    B, = drug_ids.shape
    V, D = table.shape
    half = V // 2
    n_chunks = B // (NW * CHUNK)
    idx3 = drug_ids.astype(jnp.int32).reshape(NW, n_chunks, CHUNK)
    return _sc_gather(
        idx3, table[:half], table[half:], B, D, n_chunks, half
    )


# speedup vs baseline: 1.2798x; 1.2798x over previous
import functools

import jax
import jax.numpy as jnp
from jax import lax
from jax.experimental import pallas as pl
from jax.experimental.pallas import tpu as pltpu
from jax.experimental.pallas import tpu_sc as plsc

NC = 2
NS = 16
NW = NC * NS
CHUNK = 128


def _sc_gather(idx3, table_bf, B, D, n_chunks):
    mesh = plsc.VectorSubcoreMesh(core_axis_name="c", subcore_axis_name="s")
    b_per_w = n_chunks * CHUNK
    words_per_w = b_per_w * D // 2

    @functools.partial(
        pl.kernel,
        mesh=mesh,
        out_type=jax.ShapeDtypeStruct((B, D), jnp.float32),
        compiler_params=pltpu.CompilerParams(
            use_tc_tiling_on_sc=False, needs_layout_passes=False
        ),
        scratch_types=[
            pltpu.VMEM((n_chunks, CHUNK), jnp.int32),
            pltpu.VMEM((n_chunks, CHUNK, D), jnp.bfloat16),
            pltpu.VMEM((b_per_w, D), jnp.float32),
            pltpu.SemaphoreType.DMA,
            pltpu.SemaphoreType.DMA,
        ],
    )
    def k(idx_hbm, table_hbm, out_hbm, idx_v, raw_v, rows_v, gsem, ssem):
        wid = lax.axis_index("s") * NC + lax.axis_index("c")
        base = wid * b_per_w
        pltpu.sync_copy(idx_hbm.at[wid], idx_v)
        gathers = [
            pltpu.async_copy(table_hbm.at[idx_v.at[c]], raw_v.at[c], gsem)
            for c in range(n_chunks)
        ]
        for g in gathers:
            g.wait()

        evens = lax.iota(jnp.int32, 16) * 2
        odds = evens + 1

        for c in range(n_chunks):
            def widen(i, _, c=c):
                bi = jnp.broadcast_to(c * CHUNK + i, (16,))
                for h in range(2):
                    w = plsc.bitcast(
                        raw_v[c, i, pl.ds(h * 32, 32)], jnp.int32
                    )
                    lo = plsc.bitcast(lax.shift_left(w, 16), jnp.float32)
                    hi = plsc.bitcast(
                        lax.bitwise_and(w, jnp.int32(-65536)), jnp.float32
                    )
                    plsc.store_scatter(rows_v, [bi, h * 32 + evens], lo)
                    plsc.store_scatter(rows_v, [bi, h * 32 + odds], hi)
                return ()

            lax.fori_loop(0, CHUNK, widen, ())
        pltpu.async_copy(rows_v, out_hbm.at[pl.ds(base, b_per_w)], ssem).wait()

    return k(idx3, table_bf)


def kernel(drug_ids, table):
    B, = drug_ids.shape
    _, D = table.shape
    n_chunks = B // (NW * CHUNK)
    idx3 = drug_ids.astype(jnp.int32).reshape(NW, n_chunks, CHUNK)
    table_bf = table.astype(jnp.bfloat16)
    return _sc_gather(idx3, table_bf, B, D, n_chunks)


# speedup vs baseline: 1.7228x; 1.3461x over previous
import functools

import jax
import jax.numpy as jnp
from jax import lax
from jax.experimental import pallas as pl
from jax.experimental.pallas import tpu as pltpu
from jax.experimental.pallas import tpu_sc as plsc

NC = 2
NS = 16
NW = NC * NS
CHUNK = 128


def _sc_gather(idx3, table2, B, D, n_chunks):
    mesh = plsc.VectorSubcoreMesh(core_axis_name="c", subcore_axis_name="s")
    b_per_w = n_chunks * CHUNK
    D2 = 2 * D

    @functools.partial(
        pl.kernel,
        mesh=mesh,
        out_type=jax.ShapeDtypeStruct((B, D), jnp.float32),
        scratch_types=[
            pltpu.VMEM((n_chunks, CHUNK), jnp.int32),
            pltpu.VMEM((n_chunks, CHUNK), jnp.int32),
            pltpu.VMEM((n_chunks, CHUNK, D2), jnp.float32),
            pltpu.VMEM((2, CHUNK, D), jnp.float32),
            pltpu.SemaphoreType.DMA,
            pltpu.SemaphoreType.DMA,
        ],
    )
    def k(idx_hbm, table_hbm, out_hbm, idx_v, idxp_v, raw_v, rows_v,
          gsem, ssem):
        wid = lax.axis_index("s") * NC + lax.axis_index("c")
        base = wid * b_per_w
        pltpu.sync_copy(idx_hbm.at[wid], idx_v)

        for c in range(n_chunks):
            for g in range(CHUNK // 16):
                v = idx_v[c, pl.ds(g * 16, 16)]
                idxp_v[c, pl.ds(g * 16, 16)] = lax.shift_right_logical(v, 1)

        gathers = [
            pltpu.async_copy(table_hbm.at[idxp_v.at[c]], raw_v.at[c], gsem)
            for c in range(n_chunks)
        ]
        for g in gathers:
            g.wait()

        for c in range(n_chunks):
            if c >= 2:
                pltpu.make_async_copy(
                    rows_v.at[0], out_hbm.at[pl.ds(0, CHUNK)], ssem
                ).wait()

            def pick(g, _, c=c):
                v = idx_v[c, pl.ds(g * 16, 16)]
                for j in range(16):
                    r = lax.squeeze(lax.slice_in_dim(v, j, j + 1), (0,))
                    off = (r & 1) * D
                    i = g * 16 + j
                    for q in range(D // 16):
                        rows_v[c % 2, i, pl.ds(q * 16, 16)] = raw_v[
                            c, i, pl.ds(off + q * 16, 16)
                        ]
                return ()

            lax.fori_loop(0, CHUNK // 16, pick, ())
            pltpu.async_copy(
                rows_v.at[c % 2],
                out_hbm.at[pl.ds(base + c * CHUNK, CHUNK)],
                ssem,
            )
        for _ in range(min(2, n_chunks)):
            pltpu.make_async_copy(
                rows_v.at[0], out_hbm.at[pl.ds(0, CHUNK)], ssem
            ).wait()

    return k(idx3, table2)


def kernel(drug_ids, table):
    B, = drug_ids.shape
    V, D = table.shape
    n_chunks = B // (NW * CHUNK)
    idx3 = drug_ids.astype(jnp.int32).reshape(NW, n_chunks, CHUNK)
    table2 = table.reshape(V // 2, 2 * D)
    return _sc_gather(idx3, table2, B, D, n_chunks)


# speedup vs baseline: 2.9611x; 1.7188x over previous
import functools

import jax
import jax.numpy as jnp
from jax import lax
from jax.experimental import pallas as pl
from jax.experimental.pallas import tpu as pltpu
from jax.experimental.pallas import tpu_sc as plsc

NC = 2
NS = 16
NW = NC * NS


def _sc_gather(idx2, table, B, D, b_per_w):
    mesh = plsc.VectorSubcoreMesh(core_axis_name="c", subcore_axis_name="s")

    @functools.partial(
        pl.kernel,
        mesh=mesh,
        out_type=jax.ShapeDtypeStruct((B, D), jnp.float32),
        scratch_types=[
            pltpu.VMEM((b_per_w,), jnp.int32),
            pltpu.VMEM((b_per_w, D), jnp.float32),
            pltpu.SemaphoreType.DMA,
            pltpu.SemaphoreType.DMA,
        ],
    )
    def k(idx_hbm, table_hbm, out_hbm, idx_v, rows_v, gsem, ssem):
        wid = lax.axis_index("s") * NC + lax.axis_index("c")
        base = wid * b_per_w
        pltpu.sync_copy(idx_hbm.at[wid], idx_v)

        def body(g, _):
            v = idx_v[pl.ds(g * 16, 16)]
            for j in range(16):
                r = lax.squeeze(lax.slice_in_dim(v, j, j + 1), (0,))
                pltpu.async_copy(
                    table_hbm.at[pl.ds(r, 1)],
                    rows_v.at[pl.ds(g * 16 + j, 1)],
                    gsem,
                )
            return ()

        lax.fori_loop(0, b_per_w // 16, body, ())
        pltpu.make_async_copy(
            table_hbm.at[pl.ds(0, b_per_w)], rows_v, gsem
        ).wait()
        pltpu.async_copy(rows_v, out_hbm.at[pl.ds(base, b_per_w)], ssem).wait()

    return k(idx2, table)


def kernel(drug_ids, table):
    B, = drug_ids.shape
    _, D = table.shape
    b_per_w = B // NW
    idx2 = drug_ids.astype(jnp.int32).reshape(NW, b_per_w)
    return _sc_gather(idx2, table, B, D, b_per_w)


# speedup vs baseline: 4.4298x; 1.4960x over previous
import functools

import jax
import jax.numpy as jnp
from jax import lax
from jax.experimental import pallas as pl
from jax.experimental.pallas import tpu as pltpu
from jax.experimental.pallas import tpu_sc as plsc

NC = 2
NS = 16
NW = NC * NS


def _sc_gather(idx2, table3, B, D, b_per_w):
    mesh = plsc.VectorSubcoreMesh(core_axis_name="c", subcore_axis_name="s")

    @functools.partial(
        pl.kernel,
        mesh=mesh,
        out_type=jax.ShapeDtypeStruct((B, 1, D), jnp.float32),
        scratch_types=[
            pltpu.VMEM((b_per_w,), jnp.int32),
            pltpu.VMEM((b_per_w, 1, D), jnp.float32),
            pltpu.SemaphoreType.DMA,
            pltpu.SemaphoreType.DMA,
        ],
    )
    def k(idx_hbm, table_hbm, out_hbm, idx_v, rows_v, gsem, ssem):
        wid = lax.axis_index("s") * NC + lax.axis_index("c")
        base = wid * b_per_w
        pltpu.sync_copy(idx_hbm.at[wid], idx_v)

        def body(g, _):
            v = idx_v[pl.ds(g * 16, 16)]
            for j in range(16):
                r = lax.squeeze(lax.slice_in_dim(v, j, j + 1), (0,))
                pltpu.async_copy(
                    table_hbm.at[pl.ds(r >> 3, 1), pl.ds(r & 7, 1)],
                    rows_v.at[pl.ds(g * 16 + j, 1)],
                    gsem,
                )
            return ()

        lax.fori_loop(0, b_per_w // 16, body, ())
        pltpu.make_async_copy(
            table_hbm.at[pl.ds(0, b_per_w), pl.ds(0, 1)], rows_v, gsem
        ).wait()
        pltpu.async_copy(rows_v, out_hbm.at[pl.ds(base, b_per_w)], ssem).wait()

    return k(idx2, table3)


def kernel(drug_ids, table):
    B, = drug_ids.shape
    V, D = table.shape
    b_per_w = B // NW
    idx2 = drug_ids.astype(jnp.int32).reshape(NW, b_per_w)
    table3 = table.reshape(V // 8, 8, D)
    return _sc_gather(idx2, table3, B, D, b_per_w).reshape(B, D)
